# baseline (speedup 1.0000x reference)
import functools

import jax
import jax.numpy as jnp
from jax import lax
from jax.experimental import pallas as pl
from jax.experimental.pallas import tpu as pltpu
from jax.experimental.pallas import tpu_sc as plsc

NUM_STATES = 1000000
EMBEDDING_DIM = 64
BATCH = 16384

_info = plsc.get_sparse_core_info()
_NC, _NS = _info.num_cores, _info.num_subcores
_NW = _NC * _NS
_B_PER_W = BATCH // _NW


def _make_gather():
    mesh = plsc.VectorSubcoreMesh(core_axis_name="c", subcore_axis_name="s")

    @functools.partial(
        pl.kernel,
        mesh=mesh,
        out_type=jax.ShapeDtypeStruct((BATCH, EMBEDDING_DIM), jnp.float32),
        scratch_types=[
            pltpu.VMEM((_B_PER_W,), jnp.int32),
            pltpu.VMEM((_B_PER_W, EMBEDDING_DIM), jnp.float32),
            pltpu.SemaphoreType.DMA,
        ],
        compiler_params=pltpu.CompilerParams(use_tc_tiling_on_sc=False),
    )
    def gather_kernel(table_hbm, idx_hbm, out_hbm, idx_v, rows_v, sem):
        wid = lax.axis_index("s") * _NC + lax.axis_index("c")
        base = wid * _B_PER_W
        pltpu.sync_copy(idx_hbm.at[pl.ds(base, _B_PER_W)], idx_v)
        pltpu.async_copy(table_hbm.at[idx_v], rows_v, sem).wait()
        pltpu.sync_copy(rows_v, out_hbm.at[pl.ds(base, _B_PER_W)])

    return gather_kernel


_gather = _make_gather()


def kernel(state_id, state_embedding):
    return _gather(state_embedding, state_id.astype(jnp.int32))


# speedup vs baseline: 1.0000x; 1.0000x over previous
import functools

import jax
import jax.numpy as jnp
from jax import lax
from jax.experimental import pallas as pl
from jax.experimental.pallas import tpu as pltpu
from jax.experimental.pallas import tpu_sc as plsc

NUM_STATES = 1000000
EMBEDDING_DIM = 64
BATCH = 16384

_info = plsc.get_sparse_core_info()
_NC, _NS = _info.num_cores, _info.num_subcores
_NW = _NC * _NS
_B_PER_W = BATCH // _NW


_NCHUNK = 4
_C_ROWS = _B_PER_W // _NCHUNK


def _make_gather():
    mesh = plsc.VectorSubcoreMesh(core_axis_name="c", subcore_axis_name="s")

    @functools.partial(
        pl.kernel,
        mesh=mesh,
        out_type=jax.ShapeDtypeStruct((BATCH, EMBEDDING_DIM), jnp.float32),
        scratch_types=[
            pltpu.VMEM((_NCHUNK, _C_ROWS), jnp.int32),
            pltpu.VMEM((_NCHUNK, _C_ROWS, EMBEDDING_DIM), jnp.float32),
            [pltpu.SemaphoreType.DMA] * _NCHUNK,
        ],
        compiler_params=pltpu.CompilerParams(use_tc_tiling_on_sc=False),
    )
    def gather_kernel(table_hbm, idx_hbm, out_hbm, idx_v, rows_v, sems):
        wid = lax.axis_index("s") * _NC + lax.axis_index("c")
        base = wid * _B_PER_W
        pltpu.sync_copy(idx_hbm.at[wid], idx_v)
        copies = []
        for j in range(_NCHUNK):
            copies.append(
                pltpu.async_copy(table_hbm.at[idx_v.at[j]], rows_v.at[j], sems[j])
            )
        for j in range(_NCHUNK):
            copies[j].wait()
            pltpu.sync_copy(
                rows_v.at[j], out_hbm.at[pl.ds(base + j * _C_ROWS, _C_ROWS)]
            )

    return gather_kernel


_gather = _make_gather()


def kernel(state_id, state_embedding):
    idx = state_id.astype(jnp.int32).reshape(_NW, _NCHUNK, _C_ROWS)
    return _gather(state_embedding, idx)


# speedup vs baseline: 2.4857x; 2.4856x over previous
import functools

import jax
import jax.numpy as jnp
from jax import lax
from jax.experimental import pallas as pl
from jax.experimental.pallas import tpu as pltpu
from jax.experimental.pallas import tpu_sc as plsc

NUM_STATES = 1000000
EMBEDDING_DIM = 64
BATCH = 16384

_info = plsc.get_sparse_core_info()
_NC, _NS, _L = _info.num_cores, _info.num_subcores, _info.num_lanes
_NW = _NC * _NS
_B_PER_W = BATCH // _NW
_GROUP = 8
_C_ROWS = 32
_NCHUNK = _B_PER_W // _C_ROWS


def _make_gather():
    mesh = plsc.VectorSubcoreMesh(core_axis_name="c", subcore_axis_name="s")

    @functools.partial(
        pl.kernel,
        mesh=mesh,
        out_type=jax.ShapeDtypeStruct((BATCH, EMBEDDING_DIM), jnp.float32),
        scratch_types=[
            pltpu.VMEM((_B_PER_W,), jnp.int32),
            pltpu.VMEM((2, _C_ROWS, EMBEDDING_DIM), jnp.float32),
            [pltpu.SemaphoreType.DMA] * 2,
        ],
    )
    def gather_kernel(table_hbm, idx_hbm, out_hbm, idx_v, rbuf, sems):
        wid = lax.axis_index("s") * _NC + lax.axis_index("c")
        base = wid * _B_PER_W
        pltpu.sync_copy(idx_hbm.at[wid], idx_v)

        def issue_chunk(j):
            p = j % 2
            for h in range(_C_ROWS // _L):
                v = idx_v[pl.ds(j * _C_ROWS + h * _L, _L)]
                for l in range(_L):
                    x = v[l]
                    pltpu.async_copy(
                        table_hbm.at[x >> 3, x & 7],
                        rbuf.at[p, h * _L + l],
                        sems[p],
                    )

        def drain_and_writeback(j):
            p = j % 2
            dst = out_hbm.at[pl.ds(base + j * _C_ROWS, _C_ROWS)]
            pltpu.make_async_copy(dst, rbuf.at[p], sems[p]).wait()
            pltpu.sync_copy(rbuf.at[p], dst)

        issue_chunk(0)
        for j in range(1, _NCHUNK):
            issue_chunk(j)
            drain_and_writeback(j - 1)
        drain_and_writeback(_NCHUNK - 1)

    return gather_kernel


_gather = _make_gather()


def kernel(state_id, state_embedding):
    table8 = state_embedding.reshape(NUM_STATES // _GROUP, _GROUP, EMBEDDING_DIM)
    idx = state_id.astype(jnp.int32).reshape(_NW, _B_PER_W)
    return _gather(table8, idx)
